# baseline (speedup 1.0000x reference)
import jax
import jax.numpy as jnp
from jax import lax
from jax.experimental import pallas as pl
from jax.experimental.pallas import tpu as pltpu
from jax.experimental.pallas import tpu_sc as plsc

_NB = 16384
_S = 50
_D = 32
_T = 128
_NT = _NB // _T
_NW = 32
_TPW = _NT // _NW
_SP = _S // 2


def _body(x_hbm, tab_hbm, out_hbm, xblk, idxs_v, rows_v, tblk, sem_g, sem_o):
    wid = lax.axis_index("s") * 2 + lax.axis_index("c")
    iota = lax.iota(jnp.int32, 16)
    rowsel = [g * 16 + iota for g in range(8)]

    def extract(s, sl):
        colsel = jnp.full((16,), s, dtype=jnp.int32)
        for g in range(8):
            v = plsc.load_gather(xblk, [rowsel[g], colsel])
            idxs_v[sl, pl.ds(g * 16, 16)] = v

    def fire_gather(sl):
        pltpu.async_copy(tab_hbm.at[idxs_v.at[sl]], rows_v.at[sl],
                         sem_g.at[sl])

    def wait_gather(sl):
        pltpu.make_async_copy(tab_hbm.at[idxs_v.at[sl]], rows_v.at[sl],
                              sem_g.at[sl]).wait()

    def wait_out(sl):
        pltpu.make_async_copy(tblk.at[0], out_hbm.at[0, :, 0],
                              sem_o.at[sl]).wait()

    def transpose(sl):
        rows_sl = rows_v.at[sl]
        for u in range(4):
            for ci in range(8):
                colsel = jnp.full((16,), u * 8 + ci, dtype=jnp.int32)
                for g in range(8):
                    v = plsc.load_gather(rows_sl, [rowsel[g], colsel])
                    tblk[sl, u, ci, pl.ds(g * 16, 16)] = v

    def t_body(lt, carry):
        t = wid * _TPW + lt
        pltpu.sync_copy(x_hbm.at[pl.ds(t * _T, _T), :], xblk)
        extract(0, 0)
        fire_gather(0)
        extract(1, 1)
        fire_gather(1)

        def pair(i, c2):
            s0 = 2 * i
            for j in (0, 1):
                s = s0 + j
                wait_gather(j)

                @pl.when(jnp.logical_or(i >= 1, lt > 0))
                def _():
                    wait_out(j)

                transpose(j)
                pltpu.async_copy(tblk.at[j], out_hbm.at[s, :, t],
                                 sem_o.at[j])

                @pl.when(s + 2 < _S)
                def _():
                    extract(s + 2, j)
                    fire_gather(j)

            return c2

        lax.fori_loop(0, _SP, pair, 0)
        return carry

    lax.fori_loop(0, _TPW, t_body, 0)
    wait_out(0)
    wait_out(1)


_mesh = plsc.VectorSubcoreMesh(core_axis_name="c", subcore_axis_name="s")


@jax.jit
def kernel(x, spa_emb_weight):
    out6 = pl.kernel(
        _body,
        out_type=jax.ShapeDtypeStruct((_S, _D // 8, _NT, 8, _T), jnp.float32),
        mesh=_mesh,
        scratch_types=[
            pltpu.VMEM((_T, _S), jnp.int32),
            pltpu.VMEM((2, _T), jnp.int32),
            pltpu.VMEM((2, _T, _D), jnp.float32),
            pltpu.VMEM((2, _D // 8, 8, _T), jnp.float32),
            pltpu.SemaphoreType.DMA((2,)),
            pltpu.SemaphoreType.DMA((2,)),
        ],
        compiler_params=pltpu.CompilerParams(
            use_tc_tiling_on_sc=False, needs_layout_passes=False),
    )(x, spa_emb_weight)
    return out6.transpose(2, 4, 0, 1, 3).reshape(_NB, _S, _D)


# speedup vs baseline: 1.1272x; 1.1272x over previous
import jax
import jax.numpy as jnp
from jax import lax
from jax.experimental import pallas as pl
from jax.experimental.pallas import tpu as pltpu
from jax.experimental.pallas import tpu_sc as plsc

_NB = 16384
_S = 50
_D = 32
_T = 128
_NT = _NB // _T
_NW = 32
_TPW = _NT // _NW
_SP = _S // 2


def _body(x_hbm, tab_hbm, out_hbm, xblk, idxs_v, rows_v, tblk, sem_g, sem_o):
    wid = lax.axis_index("s") * 2 + lax.axis_index("c")
    iota = lax.iota(jnp.int32, 16)
    rowsel = [g * 16 + iota for g in range(8)]

    def extract(s, sl):
        colsel = jnp.full((16,), s, dtype=jnp.int32)
        for g in range(8):
            v = plsc.load_gather(xblk, [rowsel[g], colsel])
            idxs_v[sl, pl.ds(g * 16, 16)] = v

    def fire_gather(sl):
        pltpu.async_copy(tab_hbm.at[idxs_v.at[sl]], rows_v.at[sl],
                         sem_g.at[sl])

    def wait_gather(sl):
        pltpu.make_async_copy(tab_hbm.at[idxs_v.at[sl]], rows_v.at[sl],
                              sem_g.at[sl]).wait()

    def wait_out(sl):
        pltpu.make_async_copy(tblk.at[0], out_hbm.at[0, :, 0],
                              sem_o.at[sl]).wait()

    u_half = [(iota + h * 16) // 8 for h in (0, 1)]
    in_half = [((iota + h * 16) % 8) * _T for h in (0, 1)]

    def transpose(sl):
        rows_sl = rows_v.at[sl]
        tb = tblk.at[sl]

        def b8(k8, c3):
            base = k8 * 8
            for k in range(8):
                b = base + k
                for h in (0, 1):
                    v = rows_sl[b, pl.ds(h * 16, 16)]
                    plsc.store_scatter(tb, [u_half[h], in_half[h] + b], v)
            return c3

        lax.fori_loop(0, _T // 8, b8, 0)

    def t_body(lt, carry):
        t = wid * _TPW + lt
        pltpu.sync_copy(x_hbm.at[pl.ds(t * _T, _T), :], xblk)
        extract(0, 0)
        fire_gather(0)
        extract(1, 1)
        fire_gather(1)

        def pair(i, c2):
            s0 = 2 * i
            for j in (0, 1):
                s = s0 + j
                wait_gather(j)

                @pl.when(jnp.logical_or(i >= 1, lt > 0))
                def _():
                    wait_out(j)

                transpose(j)
                pltpu.async_copy(tblk.at[j], out_hbm.at[s, :, t],
                                 sem_o.at[j])

                @pl.when(s + 2 < _S)
                def _():
                    extract(s + 2, j)
                    fire_gather(j)

            return c2

        lax.fori_loop(0, _SP, pair, 0)
        return carry

    lax.fori_loop(0, _TPW, t_body, 0)
    wait_out(0)
    wait_out(1)


_mesh = plsc.VectorSubcoreMesh(core_axis_name="c", subcore_axis_name="s")


@jax.jit
def kernel(x, spa_emb_weight):
    out6 = pl.kernel(
        _body,
        out_type=jax.ShapeDtypeStruct((_S, _D // 8, _NT, 8 * _T), jnp.float32),
        mesh=_mesh,
        scratch_types=[
            pltpu.VMEM((_T, _S), jnp.int32),
            pltpu.VMEM((2, _T), jnp.int32),
            pltpu.VMEM((2, _T, _D), jnp.float32),
            pltpu.VMEM((2, _D // 8, 8 * _T), jnp.float32),
            pltpu.SemaphoreType.DMA((2,)),
            pltpu.SemaphoreType.DMA((2,)),
        ],
        compiler_params=pltpu.CompilerParams(
            use_tc_tiling_on_sc=False, needs_layout_passes=False),
    )(x, spa_emb_weight)
    out6 = out6.reshape(_S, _D // 8, _NT, 8, _T)
    return out6.transpose(2, 4, 0, 1, 3).reshape(_NB, _S, _D)


# speedup vs baseline: 1.3121x; 1.1641x over previous
import jax
import jax.numpy as jnp
from jax import lax
from jax.experimental import pallas as pl
from jax.experimental.pallas import tpu as pltpu
from jax.experimental.pallas import tpu_sc as plsc

_NB = 16384
_S = 50
_D = 32
_T = 128
_NT = _NB // _T
_NW = 32
_TPW = _NT // _NW
_SP = _S // 2


def _body(x_hbm, tab_hbm, out_hbm, xblk, idxs_v, rows_v, tblk, sem_g, sem_o):
    wid = lax.axis_index("s") * 2 + lax.axis_index("c")
    iota = lax.iota(jnp.int32, 16)
    rowsel = [g * 16 + iota for g in range(8)]

    def extract(s, sl):
        colsel = jnp.full((16,), s, dtype=jnp.int32)
        for g in range(8):
            v = plsc.load_gather(xblk, [rowsel[g], colsel])
            idxs_v[sl, pl.ds(g * 16, 16)] = v

    def fire_gather(sl):
        pltpu.async_copy(tab_hbm.at[idxs_v.at[sl]], rows_v.at[sl],
                         sem_g.at[sl])

    def wait_gather(sl):
        pltpu.make_async_copy(tab_hbm.at[idxs_v.at[sl]], rows_v.at[sl],
                              sem_g.at[sl]).wait()

    def wait_out(sl):
        pltpu.make_async_copy(tblk.at[0], out_hbm.at[0, :, 0],
                              sem_o.at[sl]).wait()

    u_half = [(iota + h * 16) // 8 for h in (0, 1)]
    in_half = [((iota + h * 16) % 8) * _T for h in (0, 1)]

    def transpose(sl):
        rows_sl = rows_v.at[sl]
        tb = tblk.at[sl]

        @plsc.parallel_loop(0, _T, step=1, unroll=8)
        def _(b):
            for h in (0, 1):
                v = rows_sl[b, pl.ds(h * 16, 16)]
                plsc.store_scatter(tb, [u_half[h], in_half[h] + b], v)

    def t_body(lt, carry):
        t = wid * _TPW + lt
        pltpu.sync_copy(x_hbm.at[pl.ds(t * _T, _T), :], xblk)
        extract(0, 0)
        fire_gather(0)
        extract(1, 1)
        fire_gather(1)

        def pair(i, c2):
            s0 = 2 * i
            for j in (0, 1):
                s = s0 + j
                wait_gather(j)

                @pl.when(jnp.logical_or(i >= 1, lt > 0))
                def _():
                    wait_out(j)

                transpose(j)
                pltpu.async_copy(tblk.at[j], out_hbm.at[s, :, t],
                                 sem_o.at[j])

                @pl.when(s + 2 < _S)
                def _():
                    extract(s + 2, j)
                    fire_gather(j)

            return c2

        lax.fori_loop(0, _SP, pair, 0)
        return carry

    lax.fori_loop(0, _TPW, t_body, 0)
    wait_out(0)
    wait_out(1)


_mesh = plsc.VectorSubcoreMesh(core_axis_name="c", subcore_axis_name="s")


@jax.jit
def kernel(x, spa_emb_weight):
    out6 = pl.kernel(
        _body,
        out_type=jax.ShapeDtypeStruct((_S, _D // 8, _NT, 8 * _T), jnp.float32),
        mesh=_mesh,
        scratch_types=[
            pltpu.VMEM((_T, _S), jnp.int32),
            pltpu.VMEM((2, _T), jnp.int32),
            pltpu.VMEM((2, _T, _D), jnp.float32),
            pltpu.VMEM((2, _D // 8, 8 * _T), jnp.float32),
            pltpu.SemaphoreType.DMA((2,)),
            pltpu.SemaphoreType.DMA((2,)),
        ],
        compiler_params=pltpu.CompilerParams(
            use_tc_tiling_on_sc=False, needs_layout_passes=False),
    )(x, spa_emb_weight)
    out6 = out6.reshape(_S, _D // 8, _NT, 8, _T)
    return out6.transpose(2, 4, 0, 1, 3).reshape(_NB, _S, _D)


# speedup vs baseline: 1.3187x; 1.0050x over previous
import jax
import jax.numpy as jnp
from jax import lax
from jax.experimental import pallas as pl
from jax.experimental.pallas import tpu as pltpu
from jax.experimental.pallas import tpu_sc as plsc

_NB = 16384
_S = 50
_D = 32
_T = 128
_NT = _NB // _T
_NW = 32
_TPW = _NT // _NW
_SP = _S // 2


def _body(x_hbm, tab_hbm, out_hbm, xblk, idxs_v, rows_v, tblk, sem_g, sem_o):
    wid = lax.axis_index("s") * 2 + lax.axis_index("c")
    iota = lax.iota(jnp.int32, 16)
    rowsel = [g * 16 + iota for g in range(8)]

    def extract(s, sl):
        colsel = jnp.full((16,), s, dtype=jnp.int32)
        idx_sl = idxs_v.at[sl]

        @plsc.parallel_loop(0, 8, step=1, unroll=8)
        def _(g):
            v = plsc.load_gather(xblk, [g * 16 + iota, colsel])
            idx_sl[pl.ds(g * 16, 16)] = v

    def fire_gather(sl):
        pltpu.async_copy(tab_hbm.at[idxs_v.at[sl]], rows_v.at[sl],
                         sem_g.at[sl])

    def wait_gather(sl):
        pltpu.make_async_copy(tab_hbm.at[idxs_v.at[sl]], rows_v.at[sl],
                              sem_g.at[sl]).wait()

    def wait_out(sl):
        pltpu.make_async_copy(tblk.at[0], out_hbm.at[0, :, 0],
                              sem_o.at[sl]).wait()

    u_half = [(iota + h * 16) // 8 for h in (0, 1)]
    in_half = [((iota + h * 16) % 8) * _T for h in (0, 1)]

    def transpose(sl):
        rows_sl = rows_v.at[sl]
        tb = tblk.at[sl]

        @plsc.parallel_loop(0, _T, step=1, unroll=16)
        def _(b):
            for h in (0, 1):
                v = rows_sl[b, pl.ds(h * 16, 16)]
                plsc.store_scatter(tb, [u_half[h], in_half[h] + b], v)

    def t_body(lt, carry):
        t = wid * _TPW + lt
        pltpu.sync_copy(x_hbm.at[pl.ds(t * _T, _T), :], xblk)
        extract(0, 0)
        fire_gather(0)
        extract(1, 1)
        fire_gather(1)

        def pair(i, c2):
            s0 = 2 * i
            for j in (0, 1):
                s = s0 + j
                wait_gather(j)

                @pl.when(jnp.logical_or(i >= 1, lt > 0))
                def _():
                    wait_out(j)

                transpose(j)
                pltpu.async_copy(tblk.at[j], out_hbm.at[s, :, t],
                                 sem_o.at[j])

                @pl.when(s + 2 < _S)
                def _():
                    extract(s + 2, j)
                    fire_gather(j)

            return c2

        lax.fori_loop(0, _SP, pair, 0)
        return carry

    lax.fori_loop(0, _TPW, t_body, 0)
    wait_out(0)
    wait_out(1)


_mesh = plsc.VectorSubcoreMesh(core_axis_name="c", subcore_axis_name="s")


@jax.jit
def kernel(x, spa_emb_weight):
    out6 = pl.kernel(
        _body,
        out_type=jax.ShapeDtypeStruct((_S, _D // 8, _NT, 8 * _T), jnp.float32),
        mesh=_mesh,
        scratch_types=[
            pltpu.VMEM((_T, _S), jnp.int32),
            pltpu.VMEM((2, _T), jnp.int32),
            pltpu.VMEM((2, _T, _D), jnp.float32),
            pltpu.VMEM((2, _D // 8, 8 * _T), jnp.float32),
            pltpu.SemaphoreType.DMA((2,)),
            pltpu.SemaphoreType.DMA((2,)),
        ],
        compiler_params=pltpu.CompilerParams(
            use_tc_tiling_on_sc=False, needs_layout_passes=False,
            skip_device_barrier=True),
    )(x, spa_emb_weight)
    out6 = out6.reshape(_S, _D // 8, _NT, 8, _T)
    return out6.transpose(2, 4, 0, 1, 3).reshape(_NB, _S, _D)


# speedup vs baseline: 1.8032x; 1.3674x over previous
import jax
import jax.numpy as jnp
from jax import lax
from jax.experimental import pallas as pl
from jax.experimental.pallas import tpu as pltpu
from jax.experimental.pallas import tpu_sc as plsc

_NB = 16384
_S = 50
_D = 32
_T = 128
_NT = _NB // _T
_NW = 32
_TPW = _NT // _NW
_SP = _S // 2


def _body(x_hbm, tab_hbm, out_hbm, xblk, idxs_v, rows_v, tblk, sem_g, sem_o):
    wid = lax.axis_index("s") * 2 + lax.axis_index("c")
    iota = lax.iota(jnp.int32, 16)
    rowsel = [g * 16 + iota for g in range(8)]

    def extract(s, sl):
        colsel = jnp.full((16,), s, dtype=jnp.int32)
        idx_sl = idxs_v.at[sl]

        @plsc.parallel_loop(0, 8, step=1, unroll=8)
        def _(g):
            v = plsc.load_gather(xblk, [g * 16 + iota, colsel])
            idx_sl[pl.ds(g * 16, 16)] = v

    def fire_gather(sl):
        pltpu.async_copy(tab_hbm.at[idxs_v.at[sl]], rows_v.at[sl],
                         sem_g.at[sl])

    def wait_gather(sl):
        pltpu.make_async_copy(tab_hbm.at[idxs_v.at[sl]], rows_v.at[sl],
                              sem_g.at[sl]).wait()

    def wait_out(sl):
        pltpu.make_async_copy(tblk.at[0, :, :, pl.ds(0, _T)],
                              out_hbm.at[0, :, 0], sem_o.at[sl]).wait()

    u_half = [(iota + h * 16) // 8 for h in (0, 1)]
    ci_half = [(iota + h * 16) % 8 for h in (0, 1)]

    def transpose(sl):
        rows_sl = rows_v.at[sl]
        tb = tblk.at[sl]

        @plsc.parallel_loop(0, _T, step=1, unroll=16)
        def _(b):
            bs = jnp.full((16,), b, dtype=jnp.int32)
            for h in (0, 1):
                v = rows_sl[b, pl.ds(h * 16, 16)]
                plsc.store_scatter(tb, [u_half[h], ci_half[h], bs], v)

    def t_body(lt, carry):
        t = wid * _TPW + lt
        pltpu.sync_copy(x_hbm.at[pl.ds(t * _T, _T), :], xblk)
        extract(0, 0)
        fire_gather(0)
        extract(1, 1)
        fire_gather(1)

        def pair(i, c2):
            s0 = 2 * i
            for j in (0, 1):
                s = s0 + j
                wait_gather(j)

                @pl.when(jnp.logical_or(i >= 1, lt > 0))
                def _():
                    wait_out(j)

                transpose(j)
                pltpu.async_copy(tblk.at[j, :, :, pl.ds(0, _T)],
                                 out_hbm.at[s, :, t], sem_o.at[j])

                @pl.when(s + 2 < _S)
                def _():
                    extract(s + 2, j)
                    fire_gather(j)

            return c2

        lax.fori_loop(0, _SP, pair, 0)
        return carry

    lax.fori_loop(0, _TPW, t_body, 0)
    wait_out(0)
    wait_out(1)


_mesh = plsc.VectorSubcoreMesh(core_axis_name="c", subcore_axis_name="s")


@jax.jit
def kernel(x, spa_emb_weight):
    out6 = pl.kernel(
        _body,
        out_type=jax.ShapeDtypeStruct((_S, _D // 8, _NT, 8, _T), jnp.float32),
        mesh=_mesh,
        scratch_types=[
            pltpu.VMEM((_T, _S), jnp.int32),
            pltpu.VMEM((2, _T), jnp.int32),
            pltpu.VMEM((2, _T, _D), jnp.float32),
            pltpu.VMEM((2, _D // 8, 8, _T + 1), jnp.float32),
            pltpu.SemaphoreType.DMA((2,)),
            pltpu.SemaphoreType.DMA((2,)),
        ],
        compiler_params=pltpu.CompilerParams(
            use_tc_tiling_on_sc=False, needs_layout_passes=False,
            skip_device_barrier=True),
    )(x, spa_emb_weight)
    return out6.transpose(2, 4, 0, 1, 3).reshape(_NB, _S, _D)


# speedup vs baseline: 1.8035x; 1.0002x over previous
import jax
import jax.numpy as jnp
from jax import lax
from jax.experimental import pallas as pl
from jax.experimental.pallas import tpu as pltpu
from jax.experimental.pallas import tpu_sc as plsc

_NB = 16384
_S = 50
_D = 32
_T = 128
_NT = _NB // _T
_NW = 32
_TPW = _NT // _NW
_SP = _S // 2


def _body(x_hbm, tab_hbm, out_hbm, xblk, idxs_v, rows_v, tblk, sem_g, sem_o):
    wid = lax.axis_index("s") * 2 + lax.axis_index("c")
    iota = lax.iota(jnp.int32, 16)
    rowsel = [g * 16 + iota for g in range(8)]

    def extract(s, sl):
        colsel = jnp.full((16,), s, dtype=jnp.int32)
        idx_sl = idxs_v.at[sl]

        @plsc.parallel_loop(0, 8, step=1, unroll=8)
        def _(g):
            v = plsc.load_gather(xblk, [g * 16 + iota, colsel])
            idx_sl[pl.ds(g * 16, 16)] = v

    def fire_gather(sl):
        pltpu.async_copy(tab_hbm.at[idxs_v.at[sl]], rows_v.at[sl],
                         sem_g.at[sl])

    def wait_gather(sl):
        pltpu.make_async_copy(tab_hbm.at[idxs_v.at[sl]], rows_v.at[sl],
                              sem_g.at[sl]).wait()

    def wait_out(sl):
        pltpu.make_async_copy(tblk.at[0, :, :, pl.ds(0, _T)],
                              out_hbm.at[0, :, 0], sem_o.at[sl]).wait()

    u_half = [(iota + h * 16) // 8 for h in (0, 1)]
    ci_half = [(iota + h * 16) % 8 for h in (0, 1)]

    def transpose(sl):
        rows_sl = rows_v.at[sl]
        tb = tblk.at[sl]

        @plsc.parallel_loop(0, _T, step=1, unroll=16)
        def _(b):
            bs = jnp.full((16,), b, dtype=jnp.int32)
            for h in (0, 1):
                v = rows_sl[b, pl.ds(h * 16, 16)]
                plsc.store_scatter(tb, [u_half[h], ci_half[h], bs], v)

    def t_body(lt, carry):
        t = wid * _TPW + lt
        pltpu.sync_copy(x_hbm.at[pl.ds(t * _T, _T), :], xblk)
        extract(0, 0)
        fire_gather(0)
        extract(1, 1)
        fire_gather(1)

        def pair(i, c2):
            s0 = 2 * i
            for j in (0, 1):
                s = s0 + j
                wait_gather(j)

                @pl.when(jnp.logical_or(i >= 1, lt > 0))
                def _():
                    wait_out(j)

                transpose(j)
                pltpu.async_copy(tblk.at[j, :, :, pl.ds(0, _T)],
                                 out_hbm.at[s, :, t], sem_o.at[j])

                @pl.when(s + 2 < _S)
                def _():
                    extract(s + 2, j)
                    fire_gather(j)

            return c2

        lax.fori_loop(0, _SP, pair, 0)
        return carry

    lax.fori_loop(0, _TPW, t_body, 0)
    wait_out(0)
    wait_out(1)


_mesh = plsc.VectorSubcoreMesh(core_axis_name="c", subcore_axis_name="s")


@jax.jit
def kernel(x, spa_emb_weight):
    out6 = pl.kernel(
        _body,
        out_type=jax.ShapeDtypeStruct((_S, _D // 8, _NT, 8, _T), jnp.float32),
        mesh=_mesh,
        scratch_types=[
            pltpu.VMEM((_T, _S), jnp.int32),
            pltpu.VMEM((2, _T), jnp.int32),
            pltpu.VMEM((2, _T, _D), jnp.float32),
            pltpu.VMEM((2, _D // 8, 8, _T + 1), jnp.float32),
            pltpu.SemaphoreType.DMA((2,)),
            pltpu.SemaphoreType.DMA((2,)),
        ],
        compiler_params=pltpu.CompilerParams(
            use_tc_tiling_on_sc=False, needs_layout_passes=False,
            skip_device_barrier=True, disable_bounds_checks=True,
            disable_semaphore_checks=True),
    )(x, spa_emb_weight)
    return out6.transpose(2, 4, 0, 1, 3).reshape(_NB, _S, _D)


# speedup vs baseline: 1.8037x; 1.0001x over previous
import jax
import jax.numpy as jnp
from jax import lax
from jax.experimental import pallas as pl
from jax.experimental.pallas import tpu as pltpu
from jax.experimental.pallas import tpu_sc as plsc

_NB = 16384
_S = 50
_D = 32
_T = 128
_NT = _NB // _T
_NW = 32
_TPW = _NT // _NW
_SP = _S // 2


def _body(x_hbm, tab_hbm, out_hbm, xblk, idxs_v, rows_v, tblk, sem_g, sem_o):
    wid = lax.axis_index("s") * 2 + lax.axis_index("c")
    iota = lax.iota(jnp.int32, 16)
    rowsel = [g * 16 + iota for g in range(8)]

    def extract(s, sl):
        colsel = jnp.full((16,), s, dtype=jnp.int32)
        idx_sl = idxs_v.at[sl]

        @plsc.parallel_loop(0, 8, step=1, unroll=8)
        def _(g):
            v = plsc.load_gather(xblk, [g * 16 + iota, colsel])
            idx_sl[pl.ds(g * 16, 16)] = v

    def fire_gather(sl):
        pltpu.async_copy(tab_hbm.at[idxs_v.at[sl]], rows_v.at[sl],
                         sem_g.at[sl])

    def wait_gather(sl):
        pltpu.make_async_copy(tab_hbm.at[idxs_v.at[sl]], rows_v.at[sl],
                              sem_g.at[sl]).wait()

    def wait_out(sl):
        pltpu.make_async_copy(tblk.at[0, :, :, pl.ds(0, _T)],
                              out_hbm.at[0, :, 0], sem_o.at[sl]).wait()

    u_half = [(iota + h * 16) // 8 for h in (0, 1)]
    ci_half = [(iota + h * 16) % 8 for h in (0, 1)]

    def transpose(sl):
        rows_sl = rows_v.at[sl]
        tb = tblk.at[sl]

        @plsc.parallel_loop(0, _T, step=1, unroll=16)
        def _(b):
            bs = jnp.full((16,), b, dtype=jnp.int32)
            for h in (0, 1):
                v = rows_sl[b, pl.ds(h * 16, 16)]
                plsc.store_scatter(tb, [u_half[h], ci_half[h], bs], v)

    def t_body(lt, carry):
        t = wid * _TPW + lt
        pltpu.sync_copy(x_hbm.at[pl.ds(t * _T, _T), :], xblk)
        extract(0, 0)
        fire_gather(0)
        extract(1, 1)
        fire_gather(1)

        def pair(i, c2):
            s0 = 2 * i
            for j in (0, 1):
                s = s0 + j
                wait_gather(j)

                @pl.when(jnp.logical_or(i >= 1, lt > 0))
                def _():
                    wait_out(j)

                transpose(j)
                pltpu.async_copy(tblk.at[j, :, :, pl.ds(0, _T)],
                                 out_hbm.at[s, :, t], sem_o.at[j])

                @pl.when(s + 2 < _S)
                def _():
                    extract(s + 2, j)
                    fire_gather(j)

            return c2

        lax.fori_loop(0, _SP, pair, 0)
        return carry

    lax.fori_loop(0, _TPW, t_body, 0)
    wait_out(0)
    wait_out(1)


_mesh = plsc.VectorSubcoreMesh(core_axis_name="c", subcore_axis_name="s")


@jax.jit
def kernel(x, spa_emb_weight):
    out6 = pl.kernel(
        _body,
        out_type=jax.ShapeDtypeStruct((_S, _D // 8, _NT, 8, _T), jnp.float32),
        mesh=_mesh,
        scratch_types=[
            pltpu.VMEM((_T, _S), jnp.int32),
            pltpu.VMEM((2, _T), jnp.int32),
            pltpu.VMEM((2, _T, _D), jnp.float32),
            pltpu.VMEM((2, _D // 8, 8, _T + 1), jnp.float32),
            pltpu.SemaphoreType.DMA((2,)),
            pltpu.SemaphoreType.DMA((2,)),
        ],
        compiler_params=pltpu.CompilerParams(
            use_tc_tiling_on_sc=False, needs_layout_passes=False),
    )(x, spa_emb_weight)
    return out6.transpose(2, 4, 0, 1, 3).reshape(_NB, _S, _D)
